# baseline (speedup 1.0000x reference)
import functools

import jax
import jax.numpy as jnp
from jax import lax
from jax.experimental import pallas as pl
from jax.experimental.pallas import tpu as pltpu
from jax.experimental.pallas import tpu_sc as plsc

N_OP, N_T, N_C = 50000, 5000, 20000
H = 128
NB = 2048
NUM_LAYERS = 2
LANES = 16
SLAB = 128
EDGE_ALIGN = 64 * SLAB


def _pad_len(e):
    return ((e + EDGE_ALIGN - 1) // EDGE_ALIGN) * EDGE_ALIGN


def _pad_edges(ei, e_pad):
    e = ei.shape[1]
    src = jnp.concatenate([ei[0], jnp.zeros((e_pad - e,), jnp.int32)])
    dst = jnp.concatenate([ei[1], jnp.full((e_pad - e,), -1, jnp.int32)])
    return src, dst


def _dst_blocks(out_pad):
    max_rows = 9984
    nblk = -(-out_pad // max_rows)
    n128 = out_pad // 128
    blocks = []
    base = 0
    for i in range(nblk):
        size = (n128 - base // 128) // (nblk - i) * 128
        blocks.append((base, size))
        base += size
    return tuple(blocks)


def _make_agg(out_pad, rel_shapes):
    blocks = _dst_blocks(out_pad)
    zrows = ((max(b[1] for b in blocks) + 16 + 127) // 128) * 128
    zpt = zrows // 16
    trash = zrows - 8
    nrel = len(rel_shapes)
    mesh = plsc.VectorSubcoreMesh(core_axis_name="c", subcore_axis_name="s")

    out_t = jax.ShapeDtypeStruct((2 * out_pad, H), jnp.float32)
    scratch = [
        pltpu.VMEM((SLAB,), jnp.int32),
        pltpu.VMEM((SLAB,), jnp.int32),
        pltpu.VMEM((SLAB,), jnp.int32),
        pltpu.VMEM((SLAB,), jnp.int32),
        pltpu.VMEM((SLAB,), jnp.int32),
        pltpu.VMEM((SLAB,), jnp.int32),
        pltpu.VMEM((SLAB, H), jnp.float32),
        pltpu.VMEM((SLAB, H), jnp.float32),
        pltpu.VMEM((SLAB, H), jnp.float32),
        pltpu.VMEM_SHARED((zrows, H), jnp.float32),
        pltpu.SemaphoreType.DMA,
        pltpu.SemaphoreType.DMA,
        pltpu.SemaphoreType.DMA,
        pltpu.SemaphoreType.DMA,
        pltpu.SemaphoreType.DMA,
        pltpu.SemaphoreType.DMA,
    ]

    @functools.partial(pl.kernel, out_type=out_t, scratch_types=scratch,
                       mesh=mesh)
    def agg(*refs):
        zeros_hbm = refs[0]
        rel_refs = refs[1:1 + 3 * nrel]
        o_hbm = refs[1 + 3 * nrel]
        (src_st0, src_st1, dst_st0, dst_st1, adj0, adj1, rows0, rows1,
         zbuf, acc, si0, si1, sg0, sg1, ss0, ss1) = refs[2 + 3 * nrel:]
        src_st, dst_st = (src_st0, src_st1), (dst_st0, dst_st1)
        adj, rows = (adj0, adj1), (rows0, rows1)
        si, sg, ss = (si0, si1), (sg0, sg1), (ss0, ss1)
        c = lax.axis_index("c")
        s = lax.axis_index("s")
        pltpu.sync_copy(zeros_hbm, zbuf)

        def run_rel(src_hbm, dst_hbm, p_hbm, chunk, base, bsize):
            nslab = chunk // SLAB
            base_e = (c * 16 + s) * chunk

            def idx_issue(j, b):
                off = base_e + jnp.minimum(j, nslab - 1) * SLAB
                pltpu.async_copy(src_hbm.at[pl.ds(off, SLAB)],
                                 src_st[b], si[b])
                pltpu.async_copy(dst_hbm.at[pl.ds(off, SLAB)],
                                 dst_st[b], si[b])

            def idx_wait(b):
                pltpu.make_async_copy(src_hbm.at[pl.ds(0, SLAB)],
                                      src_st[b], si[b]).wait()
                pltpu.make_async_copy(dst_hbm.at[pl.ds(0, SLAB)],
                                      dst_st[b], si[b]).wait()

            def transform(b):
                for k in range(SLAB // LANES):
                    d = dst_st[b][pl.ds(k * LANES, LANES)]
                    loc = d - base
                    ok = (loc >= 0) & (loc < bsize)
                    adj[b][pl.ds(k * LANES, LANES)] = jnp.where(ok, loc,
                                                                trash)

            def head(j, b):
                idx_wait(b)
                transform(b)
                pltpu.async_copy(p_hbm.at[src_st[b]], rows[b], sg[b])

            def tail(j, b):
                pltpu.make_async_copy(p_hbm.at[src_st[b]],
                                      rows[b], sg[b]).wait()
                idx_issue(j + 2, b)
                pltpu.async_copy(rows[b], acc.at[adj[b]], ss[b], add=True)

            def scat_wait(b):
                pltpu.make_async_copy(rows[b], acc.at[adj[b]], ss[b]).wait()

            idx_issue(0, 0)
            idx_issue(1, 1)
            for b in range(2):
                head(b, b)
                tail(b, b)

            def pair(i, carry):
                for b in range(2):
                    j = 2 * i + b
                    scat_wait(b)
                    head(j, b)
                    tail(j, b)
                return carry

            lax.fori_loop(1, nslab // 2, pair, None)
            for b in range(2):
                scat_wait(b)
                idx_wait(b)

        for base, bsize in blocks:
            z0 = s * zpt
            nfull, zrem = zpt // SLAB, zpt % SLAB
            for q in range(nfull):
                pltpu.sync_copy(zbuf, acc.at[pl.ds(z0 + q * SLAB, SLAB)])
            if zrem:
                pltpu.sync_copy(zbuf.at[pl.ds(0, zrem)],
                                acc.at[pl.ds(z0 + nfull * SLAB, zrem)])
            plsc.subcore_barrier()
            for r, (e_pad, _n_src) in enumerate(rel_shapes):
                run_rel(rel_refs[3 * r + 0], rel_refs[3 * r + 1],
                        rel_refs[3 * r + 2], e_pad // 32, base, bsize)
            plsc.subcore_barrier()
            pti = bsize // 16
            obase = c * out_pad + base
            pltpu.sync_copy(acc.at[pl.ds(s * pti, pti)],
                            o_hbm.at[pl.ds(obase + s * pti, pti)])
            plsc.subcore_barrier()

    return agg


OP_PAD = 50176
_POOL_PT = OP_PAD // 32
_POOL_TAIL = _POOL_PT - (_POOL_PT // SLAB) * SLAB
_PACC = 2176


def _make_pool():
    mesh = plsc.VectorSubcoreMesh(core_axis_name="c", subcore_axis_name="s")
    out_t = [jax.ShapeDtypeStruct((2 * NB, H), jnp.float32),
             jax.ShapeDtypeStruct((2 * NB, H), jnp.float32)]
    zpt = _PACC // 16
    scratch = [
        pltpu.VMEM((SLAB,), jnp.int32),
        pltpu.VMEM((_POOL_TAIL,), jnp.int32),
        pltpu.VMEM((SLAB, H), jnp.float32),
        pltpu.VMEM((SLAB, H), jnp.float32),
        pltpu.VMEM((SLAB, H), jnp.float32),
        pltpu.VMEM_SHARED((_PACC, H), jnp.float32),
        pltpu.VMEM_SHARED((_PACC, H), jnp.float32),
        pltpu.SemaphoreType.DMA,
    ]

    @functools.partial(pl.kernel, out_type=out_t, scratch_types=scratch,
                       mesh=mesh)
    def pool(x_hbm, b_hbm, zeros_hbm, ones_hbm,
             sums, cnts,
             idx_v, idxt_v, rows, zbuf_h, ones_v,
             acc_s, acc_c, sem):
        c = lax.axis_index("c")
        s = lax.axis_index("s")
        pltpu.sync_copy(zeros_hbm, zbuf_h)
        pltpu.sync_copy(ones_hbm, ones_v)
        z0 = s * zpt
        for acc in (acc_s, acc_c):
            pltpu.sync_copy(zbuf_h, acc.at[pl.ds(z0, SLAB)])
            pltpu.sync_copy(zbuf_h.at[pl.ds(0, zpt - SLAB)],
                            acc.at[pl.ds(z0 + SLAB, zpt - SLAB)])
        plsc.subcore_barrier()
        tbase = (c * 16 + s) * _POOL_PT

        def body(j, carry):
            off = tbase + j * SLAB
            pltpu.sync_copy(b_hbm.at[pl.ds(off, SLAB)], idx_v)
            pltpu.sync_copy(x_hbm.at[pl.ds(off, SLAB)], rows)
            pltpu.sync_copy(rows, acc_s.at[idx_v], add=True)
            pltpu.sync_copy(ones_v, acc_c.at[idx_v], add=True)
            return carry

        lax.fori_loop(0, _POOL_PT // SLAB, body, None)
        toff = tbase + (_POOL_PT // SLAB) * SLAB
        pltpu.sync_copy(b_hbm.at[pl.ds(toff, _POOL_TAIL)], idxt_v)
        pltpu.sync_copy(x_hbm.at[pl.ds(toff, _POOL_TAIL)],
                        rows.at[pl.ds(0, _POOL_TAIL)])
        pltpu.sync_copy(rows.at[pl.ds(0, _POOL_TAIL)],
                        acc_s.at[idxt_v], add=True)
        pltpu.sync_copy(ones_v.at[pl.ds(0, _POOL_TAIL)],
                        acc_c.at[idxt_v], add=True)
        plsc.subcore_barrier()
        obase = c * NB + s * (NB // 16)
        pltpu.sync_copy(acc_s.at[pl.ds(s * (NB // 16), NB // 16)],
                        sums.at[pl.ds(obase, NB // 16)])
        pltpu.sync_copy(acc_c.at[pl.ds(s * (NB // 16), NB // 16)],
                        cnts.at[pl.ds(obase, NB // 16)])

    return pool


_PREC = lax.Precision.HIGHEST


def _mm_bias(x, w, b, n_out=None, bm=512):
    n, f = x.shape
    n_out = n if n_out is None else n_out

    def kfn(x_ref, w_ref, b_ref, o_ref):
        o_ref[...] = jnp.dot(x_ref[...], w_ref[...],
                             preferred_element_type=jnp.float32,
                             precision=_PREC) + b_ref[...]

    return pl.pallas_call(
        kfn,
        grid=(pl.cdiv(n_out, bm),),
        in_specs=[pl.BlockSpec((bm, f), lambda i: (i, 0)),
                  pl.BlockSpec((f, H), lambda i: (0, 0)),
                  pl.BlockSpec((1, H), lambda i: (0, 0))],
        out_specs=pl.BlockSpec((bm, H), lambda i: (i, 0)),
        out_shape=jax.ShapeDtypeStruct((n_out, H), jnp.float32),
    )(x, w, b.reshape(1, H))


def _mm_plain(x, w, bm=512):
    n, f = x.shape

    def kfn(x_ref, w_ref, o_ref):
        o_ref[...] = jnp.dot(x_ref[...], w_ref[...],
                             preferred_element_type=jnp.float32,
                             precision=_PREC)

    return pl.pallas_call(
        kfn,
        grid=(pl.cdiv(n, bm),),
        in_specs=[pl.BlockSpec((bm, f), lambda i: (i, 0)),
                  pl.BlockSpec((f, H), lambda i: (0, 0))],
        out_specs=pl.BlockSpec((bm, H), lambda i: (i, 0)),
        out_shape=jax.ShapeDtypeStruct((n, H), jnp.float32),
    )(x, w)


def _relu_root(a_all, x, w, b, n_out=None, bm=512):
    n = x.shape[0]
    n_out = n if n_out is None else n_out
    half = a_all.shape[0] // 2 // bm

    def kfn(a0_ref, a1_ref, x_ref, w_ref, b_ref, o_ref):
        o_ref[...] = jnp.maximum(
            a0_ref[...] + a1_ref[...]
            + jnp.dot(x_ref[...], w_ref[...],
                      preferred_element_type=jnp.float32,
                      precision=_PREC) + b_ref[...], 0.0)

    return pl.pallas_call(
        kfn,
        grid=(pl.cdiv(n_out, bm),),
        in_specs=[pl.BlockSpec((bm, H), lambda i: (i, 0)),
                  pl.BlockSpec((bm, H), lambda i, half=half: (i + half, 0)),
                  pl.BlockSpec((bm, H), lambda i: (i, 0)),
                  pl.BlockSpec((H, H), lambda i: (0, 0)),
                  pl.BlockSpec((1, H), lambda i: (0, 0))],
        out_specs=pl.BlockSpec((bm, H), lambda i: (i, 0)),
        out_shape=jax.ShapeDtypeStruct((n_out, H), jnp.float32),
    )(a_all, a_all, x, w, b.reshape(1, H))


def _final_heads(sums, cnts, w2, b2):
    def kfn(s_ref, c_ref, w_ref, b_ref, o_ref):
        cnt = c_ref[:NB, 0:1] + c_ref[NB:, 0:1]
        emb = (s_ref[:NB, :] + s_ref[NB:, :]) / jnp.maximum(cnt, 1.0)
        o_ref[...] = jnp.dot(emb, w_ref[...],
                             preferred_element_type=jnp.float32,
                             precision=_PREC) + b_ref[...]

    return pl.pallas_call(
        kfn,
        grid=(1,),
        in_specs=[pl.BlockSpec((2 * NB, H), lambda i: (0, 0)),
                  pl.BlockSpec((2 * NB, H), lambda i: (0, 0)),
                  pl.BlockSpec((H, H), lambda i: (0, 0)),
                  pl.BlockSpec((1, H), lambda i: (0, 0))],
        out_specs=pl.BlockSpec((NB, H), lambda i: (0, 0)),
        out_shape=jax.ShapeDtypeStruct((NB, H), jnp.float32),
    )(sums, cnts, w2, b2)


def kernel(x_operator, x_table, x_column, ei_oo, ei_to, ei_co, ei_tt, ei_cc,
           batch_operator, lin_operator_w, lin_operator_b, lin_table_w,
           lin_table_b, lin_column_w, lin_column_b,
           w_rel_oo, b_rel_oo, w_root_oo, w_rel_to, b_rel_to, w_root_to,
           w_rel_co, b_rel_co, w_root_co, w_rel_tt, b_rel_tt, w_root_tt,
           w_rel_cc, b_rel_cc, w_root_cc,
           lin_mem_w, lin_mem_b, lin_time_w, lin_time_b):
    e_oo_p = _pad_len(ei_oo.shape[1])
    e_to_p = _pad_len(ei_to.shape[1])
    e_co_p = _pad_len(ei_co.shape[1])
    e_tt_p = _pad_len(ei_tt.shape[1])
    e_cc_p = _pad_len(ei_cc.shape[1])
    src_oo, dst_oo = _pad_edges(ei_oo, e_oo_p)
    src_to, dst_to = _pad_edges(ei_to, e_to_p)
    src_co, dst_co = _pad_edges(ei_co, e_co_p)
    src_tt, dst_tt = _pad_edges(ei_tt, e_tt_p)
    src_cc, dst_cc = _pad_edges(ei_cc, e_cc_p)

    zeros_h = jnp.zeros((SLAB, H), jnp.float32)
    ones_h = jnp.ones((SLAB, H), jnp.float32)
    batch_pad = jnp.concatenate(
        [batch_operator, jnp.full((OP_PAD - N_OP,), NB, jnp.int32)])

    w_root_op = w_root_oo + w_root_to + w_root_co
    b_op = b_rel_oo + b_rel_to + b_rel_co

    t_pad, c_pad = 5120, 20480
    agg_op = _make_agg(OP_PAD, ((e_oo_p, N_OP), (e_to_p, N_T), (e_co_p, N_C)))
    agg_t = _make_agg(t_pad, ((e_tt_p, N_T),))
    agg_c = _make_agg(c_pad, ((e_cc_p, N_C),))
    pool = _make_pool()

    x_op = _mm_bias(x_operator, lin_operator_w, lin_operator_b, n_out=OP_PAD)
    x_t = _mm_bias(x_table, lin_table_w, lin_table_b)
    x_c = _mm_bias(x_column, lin_column_w, lin_column_b)

    for _ in range(NUM_LAYERS):
        poo = _mm_plain(x_op, w_rel_oo)
        pto = _mm_plain(x_t, w_rel_to)
        pco = _mm_plain(x_c, w_rel_co)
        ptt = _mm_plain(x_t, w_rel_tt)
        pcc = _mm_plain(x_c, w_rel_cc)
        a_op = agg_op(zeros_h, src_oo, dst_oo, poo,
                      src_to, dst_to, pto,
                      src_co, dst_co, pco)
        a_t = agg_t(zeros_h, src_tt, dst_tt, ptt)
        a_c = agg_c(zeros_h, src_cc, dst_cc, pcc)
        x_op = _relu_root(a_op, x_op, w_root_op, b_op)
        x_t = _relu_root(a_t, x_t, w_root_tt, b_rel_tt)
        x_c = _relu_root(a_c, x_c, w_root_cc, b_rel_cc)

    sums, cnts = pool(x_op, batch_pad, zeros_h, ones_h)

    w2 = jnp.zeros((H, H), jnp.float32)
    w2 = w2.at[:, 0:1].set(lin_mem_w).at[:, 1:2].set(lin_time_w)
    b2 = jnp.zeros((1, H), jnp.float32)
    b2 = b2.at[0, 0].set(lin_mem_b[0]).at[0, 1].set(lin_time_b[0])
    out = _final_heads(sums, cnts, w2, b2)
    return (out[:, 0], out[:, 1])


# speedup vs baseline: 1.1267x; 1.1267x over previous
import functools

import jax
import jax.numpy as jnp
from jax import lax
from jax.experimental import pallas as pl
from jax.experimental.pallas import tpu as pltpu
from jax.experimental.pallas import tpu_sc as plsc

N_OP, N_T, N_C = 50000, 5000, 20000
H = 128
NB = 2048
NUM_LAYERS = 2
LANES = 16
SLAB = 128
EDGE_ALIGN = 64 * SLAB


def _pad_len(e):
    return ((e + EDGE_ALIGN - 1) // EDGE_ALIGN) * EDGE_ALIGN


def _pad_edges(ei, e_pad):
    e = ei.shape[1]
    src = jnp.concatenate([ei[0], jnp.zeros((e_pad - e,), jnp.int32)])
    dst = jnp.concatenate([ei[1], jnp.full((e_pad - e,), -1, jnp.int32)])
    return src, dst


def _dst_blocks(out_pad):
    max_rows = 10112
    nblk = -(-out_pad // max_rows)
    n128 = out_pad // 128
    blocks = []
    base = 0
    for i in range(nblk):
        size = (n128 - base // 128) // (nblk - i) * 128
        blocks.append((base, size))
        base += size
    return tuple(blocks)


def _make_agg(out_pad, rel_shapes):
    blocks = _dst_blocks(out_pad)
    zrows = ((max(b[1] for b in blocks) + 16 + 127) // 128) * 128
    zpt = zrows // 16
    trash = zrows - 8
    nrel = len(rel_shapes)
    chmax = max(e // 32 for e, _ in rel_shapes)
    mesh = plsc.VectorSubcoreMesh(core_axis_name="c", subcore_axis_name="s")

    out_t = jax.ShapeDtypeStruct((2 * out_pad, H), jnp.float32)
    scratch = [
        pltpu.VMEM((chmax,), jnp.int32),
        pltpu.VMEM((chmax,), jnp.int32),
        pltpu.VMEM((SLAB,), jnp.int32),
        pltpu.VMEM((SLAB,), jnp.int32),
        pltpu.VMEM((SLAB, H), jnp.float32),
        pltpu.VMEM((SLAB, H), jnp.float32),
        pltpu.VMEM((16, H), jnp.float32),
        pltpu.VMEM_SHARED((zrows, H), jnp.float32),
        pltpu.SemaphoreType.DMA,
        pltpu.SemaphoreType.DMA,
        pltpu.SemaphoreType.DMA,
        pltpu.SemaphoreType.DMA,
    ]

    @functools.partial(pl.kernel, out_type=out_t, scratch_types=scratch,
                       mesh=mesh)
    def agg(*refs):
        zeros_hbm = refs[0]
        rel_refs = refs[1:1 + 3 * nrel]
        o_hbm = refs[1 + 3 * nrel]
        (src_ch, dst_ch, adj0, adj1, rows0, rows1, zbuf, acc,
         sg0, sg1, ss0, ss1) = refs[2 + 3 * nrel:]
        adjb, rows = (adj0, adj1), (rows0, rows1)
        sg, ss = (sg0, sg1), (ss0, ss1)
        c = lax.axis_index("c")
        s = lax.axis_index("s")
        pltpu.sync_copy(zeros_hbm, zbuf)

        def run_rel(src_hbm, dst_hbm, p_hbm, chunk, base, bsize):
            nslab = chunk // SLAB
            base_e = (c * 16 + s) * chunk
            pltpu.sync_copy(src_hbm.at[pl.ds(base_e, chunk)],
                            src_ch.at[pl.ds(0, chunk)])
            pltpu.sync_copy(dst_hbm.at[pl.ds(base_e, chunk)],
                            dst_ch.at[pl.ds(0, chunk)])

            def prep(j, b):
                for k in range(SLAB // LANES):
                    dv = dst_ch[pl.ds(j * SLAB + k * LANES, LANES)]
                    loc = dv - base
                    ok = (loc >= 0) & (loc < bsize)
                    adjb[b][pl.ds(k * LANES, LANES)] = jnp.where(ok, loc,
                                                                 trash)

            def pair(i, carry):
                gh, sh = [], []
                for b in range(2):
                    j = 2 * i + b
                    prep(j, b)
                    gh.append(pltpu.async_copy(
                        p_hbm.at[src_ch.at[pl.ds(j * SLAB, SLAB)]],
                        rows[b], sg[b]))
                for b in range(2):
                    gh[b].wait()
                    sh.append(pltpu.async_copy(rows[b], acc.at[adjb[b]],
                                               ss[b], add=True))
                for b in range(2):
                    sh[b].wait()
                return carry

            lax.fori_loop(0, nslab // 2, pair, None)

        for base, bsize in blocks:
            z0 = s * zpt
            nfull, zrem = zpt // 16, zpt % 16
            for q in range(nfull):
                pltpu.sync_copy(zbuf, acc.at[pl.ds(z0 + q * 16, 16)])
            if zrem:
                pltpu.sync_copy(zbuf.at[pl.ds(0, zrem)],
                                acc.at[pl.ds(z0 + nfull * 16, zrem)])
            plsc.subcore_barrier()
            for r, (e_pad, _n_src) in enumerate(rel_shapes):
                run_rel(rel_refs[3 * r + 0], rel_refs[3 * r + 1],
                        rel_refs[3 * r + 2], e_pad // 32, base, bsize)
            plsc.subcore_barrier()
            pti = bsize // 16
            obase = c * out_pad + base
            pltpu.sync_copy(acc.at[pl.ds(s * pti, pti)],
                            o_hbm.at[pl.ds(obase + s * pti, pti)])
            plsc.subcore_barrier()

    return agg


OP_PAD = 50176
_POOL_PT = OP_PAD // 32
_POOL_TAIL = _POOL_PT - (_POOL_PT // SLAB) * SLAB
_PACC = 2176


def _make_pool():
    mesh = plsc.VectorSubcoreMesh(core_axis_name="c", subcore_axis_name="s")
    out_t = [jax.ShapeDtypeStruct((2 * NB, H), jnp.float32),
             jax.ShapeDtypeStruct((2 * NB, H), jnp.float32)]
    zpt = _PACC // 16
    scratch = [
        pltpu.VMEM((SLAB,), jnp.int32),
        pltpu.VMEM((_POOL_TAIL,), jnp.int32),
        pltpu.VMEM((SLAB, H), jnp.float32),
        pltpu.VMEM((SLAB, H), jnp.float32),
        pltpu.VMEM((SLAB, H), jnp.float32),
        pltpu.VMEM_SHARED((_PACC, H), jnp.float32),
        pltpu.VMEM_SHARED((_PACC, H), jnp.float32),
        pltpu.SemaphoreType.DMA,
    ]

    @functools.partial(pl.kernel, out_type=out_t, scratch_types=scratch,
                       mesh=mesh)
    def pool(x_hbm, b_hbm, zeros_hbm, ones_hbm,
             sums, cnts,
             idx_v, idxt_v, rows, zbuf_h, ones_v,
             acc_s, acc_c, sem):
        c = lax.axis_index("c")
        s = lax.axis_index("s")
        pltpu.sync_copy(zeros_hbm, zbuf_h)
        pltpu.sync_copy(ones_hbm, ones_v)
        z0 = s * zpt
        for acc in (acc_s, acc_c):
            pltpu.sync_copy(zbuf_h, acc.at[pl.ds(z0, SLAB)])
            pltpu.sync_copy(zbuf_h.at[pl.ds(0, zpt - SLAB)],
                            acc.at[pl.ds(z0 + SLAB, zpt - SLAB)])
        plsc.subcore_barrier()
        tbase = (c * 16 + s) * _POOL_PT

        def body(j, carry):
            off = tbase + j * SLAB
            pltpu.sync_copy(b_hbm.at[pl.ds(off, SLAB)], idx_v)
            pltpu.sync_copy(x_hbm.at[pl.ds(off, SLAB)], rows)
            pltpu.sync_copy(rows, acc_s.at[idx_v], add=True)
            pltpu.sync_copy(ones_v, acc_c.at[idx_v], add=True)
            return carry

        lax.fori_loop(0, _POOL_PT // SLAB, body, None)
        toff = tbase + (_POOL_PT // SLAB) * SLAB
        pltpu.sync_copy(b_hbm.at[pl.ds(toff, _POOL_TAIL)], idxt_v)
        pltpu.sync_copy(x_hbm.at[pl.ds(toff, _POOL_TAIL)],
                        rows.at[pl.ds(0, _POOL_TAIL)])
        pltpu.sync_copy(rows.at[pl.ds(0, _POOL_TAIL)],
                        acc_s.at[idxt_v], add=True)
        pltpu.sync_copy(ones_v.at[pl.ds(0, _POOL_TAIL)],
                        acc_c.at[idxt_v], add=True)
        plsc.subcore_barrier()
        obase = c * NB + s * (NB // 16)
        pltpu.sync_copy(acc_s.at[pl.ds(s * (NB // 16), NB // 16)],
                        sums.at[pl.ds(obase, NB // 16)])
        pltpu.sync_copy(acc_c.at[pl.ds(s * (NB // 16), NB // 16)],
                        cnts.at[pl.ds(obase, NB // 16)])

    return pool


_PREC = lax.Precision.HIGHEST


def _mm_bias(x, w, b, n_out=None, bm=512):
    n, f = x.shape
    n_out = n if n_out is None else n_out

    def kfn(x_ref, w_ref, b_ref, o_ref):
        o_ref[...] = jnp.dot(x_ref[...], w_ref[...],
                             preferred_element_type=jnp.float32,
                             precision=_PREC) + b_ref[...]

    return pl.pallas_call(
        kfn,
        grid=(pl.cdiv(n_out, bm),),
        in_specs=[pl.BlockSpec((bm, f), lambda i: (i, 0)),
                  pl.BlockSpec((f, H), lambda i: (0, 0)),
                  pl.BlockSpec((1, H), lambda i: (0, 0))],
        out_specs=pl.BlockSpec((bm, H), lambda i: (i, 0)),
        out_shape=jax.ShapeDtypeStruct((n_out, H), jnp.float32),
    )(x, w, b.reshape(1, H))


def _mm_plain(x, w, bm=512):
    n, f = x.shape

    def kfn(x_ref, w_ref, o_ref):
        o_ref[...] = jnp.dot(x_ref[...], w_ref[...],
                             preferred_element_type=jnp.float32,
                             precision=_PREC)

    return pl.pallas_call(
        kfn,
        grid=(pl.cdiv(n, bm),),
        in_specs=[pl.BlockSpec((bm, f), lambda i: (i, 0)),
                  pl.BlockSpec((f, H), lambda i: (0, 0))],
        out_specs=pl.BlockSpec((bm, H), lambda i: (i, 0)),
        out_shape=jax.ShapeDtypeStruct((n, H), jnp.float32),
    )(x, w)


def _relu_root(a_all, x, w, b, n_out=None, bm=512):
    n = x.shape[0]
    n_out = n if n_out is None else n_out
    half = a_all.shape[0] // 2 // bm

    def kfn(a0_ref, a1_ref, x_ref, w_ref, b_ref, o_ref):
        o_ref[...] = jnp.maximum(
            a0_ref[...] + a1_ref[...]
            + jnp.dot(x_ref[...], w_ref[...],
                      preferred_element_type=jnp.float32,
                      precision=_PREC) + b_ref[...], 0.0)

    return pl.pallas_call(
        kfn,
        grid=(pl.cdiv(n_out, bm),),
        in_specs=[pl.BlockSpec((bm, H), lambda i: (i, 0)),
                  pl.BlockSpec((bm, H), lambda i, half=half: (i + half, 0)),
                  pl.BlockSpec((bm, H), lambda i: (i, 0)),
                  pl.BlockSpec((H, H), lambda i: (0, 0)),
                  pl.BlockSpec((1, H), lambda i: (0, 0))],
        out_specs=pl.BlockSpec((bm, H), lambda i: (i, 0)),
        out_shape=jax.ShapeDtypeStruct((n_out, H), jnp.float32),
    )(a_all, a_all, x, w, b.reshape(1, H))


def _final_heads(sums, cnts, w2, b2):
    def kfn(s_ref, c_ref, w_ref, b_ref, o_ref):
        cnt = c_ref[:NB, 0:1] + c_ref[NB:, 0:1]
        emb = (s_ref[:NB, :] + s_ref[NB:, :]) / jnp.maximum(cnt, 1.0)
        o_ref[...] = jnp.dot(emb, w_ref[...],
                             preferred_element_type=jnp.float32,
                             precision=_PREC) + b_ref[...]

    return pl.pallas_call(
        kfn,
        grid=(1,),
        in_specs=[pl.BlockSpec((2 * NB, H), lambda i: (0, 0)),
                  pl.BlockSpec((2 * NB, H), lambda i: (0, 0)),
                  pl.BlockSpec((H, H), lambda i: (0, 0)),
                  pl.BlockSpec((1, H), lambda i: (0, 0))],
        out_specs=pl.BlockSpec((NB, H), lambda i: (0, 0)),
        out_shape=jax.ShapeDtypeStruct((NB, H), jnp.float32),
    )(sums, cnts, w2, b2)


def kernel(x_operator, x_table, x_column, ei_oo, ei_to, ei_co, ei_tt, ei_cc,
           batch_operator, lin_operator_w, lin_operator_b, lin_table_w,
           lin_table_b, lin_column_w, lin_column_b,
           w_rel_oo, b_rel_oo, w_root_oo, w_rel_to, b_rel_to, w_root_to,
           w_rel_co, b_rel_co, w_root_co, w_rel_tt, b_rel_tt, w_root_tt,
           w_rel_cc, b_rel_cc, w_root_cc,
           lin_mem_w, lin_mem_b, lin_time_w, lin_time_b):
    e_oo_p = _pad_len(ei_oo.shape[1])
    e_to_p = _pad_len(ei_to.shape[1])
    e_co_p = _pad_len(ei_co.shape[1])
    e_tt_p = _pad_len(ei_tt.shape[1])
    e_cc_p = _pad_len(ei_cc.shape[1])
    src_oo, dst_oo = _pad_edges(ei_oo, e_oo_p)
    src_to, dst_to = _pad_edges(ei_to, e_to_p)
    src_co, dst_co = _pad_edges(ei_co, e_co_p)
    src_tt, dst_tt = _pad_edges(ei_tt, e_tt_p)
    src_cc, dst_cc = _pad_edges(ei_cc, e_cc_p)

    zeros_h = jnp.zeros((SLAB, H), jnp.float32)
    ones_h = jnp.ones((SLAB, H), jnp.float32)
    batch_pad = jnp.concatenate(
        [batch_operator, jnp.full((OP_PAD - N_OP,), NB, jnp.int32)])

    w_root_op = w_root_oo + w_root_to + w_root_co
    b_op = b_rel_oo + b_rel_to + b_rel_co

    t_pad, c_pad = 5120, 20480
    agg_op = _make_agg(OP_PAD, ((e_oo_p, N_OP), (e_to_p, N_T), (e_co_p, N_C)))
    agg_t = _make_agg(t_pad, ((e_tt_p, N_T),))
    agg_c = _make_agg(c_pad, ((e_cc_p, N_C),))
    zeros_16 = jnp.zeros((16, H), jnp.float32)
    pool = _make_pool()

    x_op = _mm_bias(x_operator, lin_operator_w, lin_operator_b, n_out=OP_PAD)
    x_t = _mm_bias(x_table, lin_table_w, lin_table_b)
    x_c = _mm_bias(x_column, lin_column_w, lin_column_b)

    for _ in range(NUM_LAYERS):
        poo = _mm_plain(x_op, w_rel_oo)
        pto = _mm_plain(x_t, w_rel_to)
        pco = _mm_plain(x_c, w_rel_co)
        ptt = _mm_plain(x_t, w_rel_tt)
        pcc = _mm_plain(x_c, w_rel_cc)
        a_op = agg_op(zeros_16, src_oo, dst_oo, poo,
                      src_to, dst_to, pto,
                      src_co, dst_co, pco)
        a_t = agg_t(zeros_16, src_tt, dst_tt, ptt)
        a_c = agg_c(zeros_16, src_cc, dst_cc, pcc)
        x_op = _relu_root(a_op, x_op, w_root_op, b_op)
        x_t = _relu_root(a_t, x_t, w_root_tt, b_rel_tt)
        x_c = _relu_root(a_c, x_c, w_root_cc, b_rel_cc)

    sums, cnts = pool(x_op, batch_pad, zeros_h, ones_h)

    w2 = jnp.zeros((H, H), jnp.float32)
    w2 = w2.at[:, 0:1].set(lin_mem_w).at[:, 1:2].set(lin_time_w)
    b2 = jnp.zeros((1, H), jnp.float32)
    b2 = b2.at[0, 0].set(lin_mem_b[0]).at[0, 1].set(lin_time_b[0])
    out = _final_heads(sums, cnts, w2, b2)
    return (out[:, 0], out[:, 1])


# speedup vs baseline: 2.2683x; 2.0132x over previous
import functools

import jax
import jax.numpy as jnp
from jax import lax
from jax.experimental import pallas as pl
from jax.experimental.pallas import tpu as pltpu
from jax.experimental.pallas import tpu_sc as plsc

N_OP, N_T, N_C = 50000, 5000, 20000
H = 128
NB = 2048
NUM_LAYERS = 2
LANES = 16
SLAB = 128
EDGE_ALIGN = 32 * SLAB


def _pad_len(e):
    return ((e + EDGE_ALIGN - 1) // EDGE_ALIGN) * EDGE_ALIGN


def _pad_edges(ei, e_pad):
    e = ei.shape[1]
    src = jnp.concatenate([ei[0], jnp.zeros((e_pad - e,), jnp.int32)])
    dst = jnp.concatenate([ei[1], jnp.full((e_pad - e,), -1, jnp.int32)])
    return src, dst


def _dst_blocks(out_pad):
    max_rows = 12544
    nblk = -(-out_pad // max_rows)
    n128 = out_pad // 128
    blocks = []
    base = 0
    for i in range(nblk):
        size = (n128 - base // 128) // (nblk - i) * 128
        blocks.append((base, size))
        base += size
    return tuple(blocks)


def _make_agg(out_pad, rel_shapes):
    blocks = _dst_blocks(out_pad)
    zrows = ((max(b[1] for b in blocks) + 16 + 127) // 128) * 128
    zpt = zrows // 16
    trash = zrows - 8
    nrel = len(rel_shapes)
    chmax = max(e // 32 for e, _ in rel_shapes)
    mesh = plsc.VectorSubcoreMesh(core_axis_name="c", subcore_axis_name="s")

    out_t = jax.ShapeDtypeStruct((2 * out_pad, H), jnp.float32)
    scratch = [
        pltpu.VMEM((SLAB,), jnp.int32),
        pltpu.VMEM((SLAB,), jnp.int32),
        pltpu.VMEM((SLAB,), jnp.int32),
        pltpu.VMEM((SLAB, H), jnp.float32),
        pltpu.VMEM((16, H), jnp.float32),
        pltpu.VMEM_SHARED((zrows, H), jnp.float32),
        pltpu.SemaphoreType.DMA,
    ]

    @functools.partial(pl.kernel, out_type=out_t, scratch_types=scratch,
                       mesh=mesh)
    def agg(*refs):
        zeros_hbm = refs[0]
        rel_refs = refs[1:1 + 3 * nrel]
        o_hbm = refs[1 + 3 * nrel]
        (src_st, dst_st, dst_adj, rows, zbuf, acc, sem) = refs[2 + 3 * nrel:]
        c = lax.axis_index("c")
        s = lax.axis_index("s")
        trash_t = trash - 8 * s
        pltpu.sync_copy(zeros_hbm, zbuf)

        def run_rel(src_hbm, dst_hbm, p_hbm, chunk, base, bsize):
            nslab = chunk // SLAB
            base_e = (c * 16 + s) * chunk

            def body(j, carry, src_hbm=src_hbm, dst_hbm=dst_hbm,
                     p_hbm=p_hbm, chunk=chunk, base=base, bsize=bsize):
                off = base_e + j * SLAB
                pltpu.sync_copy(src_hbm.at[pl.ds(off, SLAB)], src_st)
                pltpu.sync_copy(dst_hbm.at[pl.ds(off, SLAB)], dst_st)
                for k in range(SLAB // LANES):
                    d = dst_st[pl.ds(k * LANES, LANES)]
                    loc = d - base
                    ok = (loc >= 0) & (loc < bsize)
                    dst_adj[pl.ds(k * LANES, LANES)] = jnp.where(ok, loc,
                                                                 trash_t)
                pltpu.async_copy(p_hbm.at[src_st], rows, sem).wait()
                pltpu.sync_copy(rows, acc.at[dst_adj], add=True)
                return carry

            lax.fori_loop(0, nslab, body, None)

        for base, bsize in blocks:
            z0 = s * zpt
            nfull, zrem = zpt // 16, zpt % 16
            for q in range(nfull):
                pltpu.sync_copy(zbuf, acc.at[pl.ds(z0 + q * 16, 16)])
            if zrem:
                pltpu.sync_copy(zbuf.at[pl.ds(0, zrem)],
                                acc.at[pl.ds(z0 + nfull * 16, zrem)])
            plsc.subcore_barrier()
            for r, (e_pad, _n_src) in enumerate(rel_shapes):
                run_rel(rel_refs[3 * r + 0], rel_refs[3 * r + 1],
                        rel_refs[3 * r + 2], e_pad // 32, base, bsize)
            plsc.subcore_barrier()
            pti = bsize // 16
            obase = c * out_pad + base
            pltpu.sync_copy(acc.at[pl.ds(s * pti, pti)],
                            o_hbm.at[pl.ds(obase + s * pti, pti)])
            plsc.subcore_barrier()

    return agg


OP_PAD = 50176
_POOL_PT = OP_PAD // 32
_POOL_TAIL = _POOL_PT - (_POOL_PT // SLAB) * SLAB
_PACC = 2176


def _make_pool():
    mesh = plsc.VectorSubcoreMesh(core_axis_name="c", subcore_axis_name="s")
    out_t = [jax.ShapeDtypeStruct((2 * NB, H), jnp.float32),
             jax.ShapeDtypeStruct((2 * NB, H), jnp.float32)]
    zpt = _PACC // 16
    scratch = [
        pltpu.VMEM((SLAB,), jnp.int32),
        pltpu.VMEM((_POOL_TAIL,), jnp.int32),
        pltpu.VMEM((SLAB, H), jnp.float32),
        pltpu.VMEM((SLAB, H), jnp.float32),
        pltpu.VMEM((SLAB, H), jnp.float32),
        pltpu.VMEM_SHARED((_PACC, H), jnp.float32),
        pltpu.VMEM_SHARED((_PACC, H), jnp.float32),
        pltpu.SemaphoreType.DMA,
    ]

    @functools.partial(pl.kernel, out_type=out_t, scratch_types=scratch,
                       mesh=mesh)
    def pool(x_hbm, b_hbm, zeros_hbm, ones_hbm,
             sums, cnts,
             idx_v, idxt_v, rows, zbuf_h, ones_v,
             acc_s, acc_c, sem):
        c = lax.axis_index("c")
        s = lax.axis_index("s")
        pltpu.sync_copy(zeros_hbm, zbuf_h)
        pltpu.sync_copy(ones_hbm, ones_v)
        z0 = s * zpt
        for acc in (acc_s, acc_c):
            pltpu.sync_copy(zbuf_h, acc.at[pl.ds(z0, SLAB)])
            pltpu.sync_copy(zbuf_h.at[pl.ds(0, zpt - SLAB)],
                            acc.at[pl.ds(z0 + SLAB, zpt - SLAB)])
        plsc.subcore_barrier()
        tbase = (c * 16 + s) * _POOL_PT

        def body(j, carry):
            off = tbase + j * SLAB
            pltpu.sync_copy(b_hbm.at[pl.ds(off, SLAB)], idx_v)
            pltpu.sync_copy(x_hbm.at[pl.ds(off, SLAB)], rows)
            pltpu.sync_copy(rows, acc_s.at[idx_v], add=True)
            pltpu.sync_copy(ones_v, acc_c.at[idx_v], add=True)
            return carry

        lax.fori_loop(0, _POOL_PT // SLAB, body, None)
        toff = tbase + (_POOL_PT // SLAB) * SLAB
        pltpu.sync_copy(b_hbm.at[pl.ds(toff, _POOL_TAIL)], idxt_v)
        pltpu.sync_copy(x_hbm.at[pl.ds(toff, _POOL_TAIL)],
                        rows.at[pl.ds(0, _POOL_TAIL)])
        pltpu.sync_copy(rows.at[pl.ds(0, _POOL_TAIL)],
                        acc_s.at[idxt_v], add=True)
        pltpu.sync_copy(ones_v.at[pl.ds(0, _POOL_TAIL)],
                        acc_c.at[idxt_v], add=True)
        plsc.subcore_barrier()
        obase = c * NB + s * (NB // 16)
        pltpu.sync_copy(acc_s.at[pl.ds(s * (NB // 16), NB // 16)],
                        sums.at[pl.ds(obase, NB // 16)])
        pltpu.sync_copy(acc_c.at[pl.ds(s * (NB // 16), NB // 16)],
                        cnts.at[pl.ds(obase, NB // 16)])

    return pool


_PREC = lax.Precision.HIGHEST


def _mm_bias(x, w, b, n_out=None, bm=512):
    n, f = x.shape
    n_out = n if n_out is None else n_out

    def kfn(x_ref, w_ref, b_ref, o_ref):
        o_ref[...] = jnp.dot(x_ref[...], w_ref[...],
                             preferred_element_type=jnp.float32,
                             precision=_PREC) + b_ref[...]

    return pl.pallas_call(
        kfn,
        grid=(pl.cdiv(n_out, bm),),
        in_specs=[pl.BlockSpec((bm, f), lambda i: (i, 0)),
                  pl.BlockSpec((f, H), lambda i: (0, 0)),
                  pl.BlockSpec((1, H), lambda i: (0, 0))],
        out_specs=pl.BlockSpec((bm, H), lambda i: (i, 0)),
        out_shape=jax.ShapeDtypeStruct((n_out, H), jnp.float32),
    )(x, w, b.reshape(1, H))


def _mm_plain(x, w, bm=512):
    n, f = x.shape

    def kfn(x_ref, w_ref, o_ref):
        o_ref[...] = jnp.dot(x_ref[...], w_ref[...],
                             preferred_element_type=jnp.float32,
                             precision=_PREC)

    return pl.pallas_call(
        kfn,
        grid=(pl.cdiv(n, bm),),
        in_specs=[pl.BlockSpec((bm, f), lambda i: (i, 0)),
                  pl.BlockSpec((f, H), lambda i: (0, 0))],
        out_specs=pl.BlockSpec((bm, H), lambda i: (i, 0)),
        out_shape=jax.ShapeDtypeStruct((n, H), jnp.float32),
    )(x, w)


def _relu_root(a_all, x, w, b, n_out=None, bm=512):
    n = x.shape[0]
    n_out = n if n_out is None else n_out
    half = a_all.shape[0] // 2 // bm

    def kfn(a0_ref, a1_ref, x_ref, w_ref, b_ref, o_ref):
        o_ref[...] = jnp.maximum(
            a0_ref[...] + a1_ref[...]
            + jnp.dot(x_ref[...], w_ref[...],
                      preferred_element_type=jnp.float32,
                      precision=_PREC) + b_ref[...], 0.0)

    return pl.pallas_call(
        kfn,
        grid=(pl.cdiv(n_out, bm),),
        in_specs=[pl.BlockSpec((bm, H), lambda i: (i, 0)),
                  pl.BlockSpec((bm, H), lambda i, half=half: (i + half, 0)),
                  pl.BlockSpec((bm, H), lambda i: (i, 0)),
                  pl.BlockSpec((H, H), lambda i: (0, 0)),
                  pl.BlockSpec((1, H), lambda i: (0, 0))],
        out_specs=pl.BlockSpec((bm, H), lambda i: (i, 0)),
        out_shape=jax.ShapeDtypeStruct((n_out, H), jnp.float32),
    )(a_all, a_all, x, w, b.reshape(1, H))


def _final_heads(sums, cnts, w2, b2):
    def kfn(s_ref, c_ref, w_ref, b_ref, o_ref):
        cnt = c_ref[:NB, 0:1] + c_ref[NB:, 0:1]
        emb = (s_ref[:NB, :] + s_ref[NB:, :]) / jnp.maximum(cnt, 1.0)
        o_ref[...] = jnp.dot(emb, w_ref[...],
                             preferred_element_type=jnp.float32,
                             precision=_PREC) + b_ref[...]

    return pl.pallas_call(
        kfn,
        grid=(1,),
        in_specs=[pl.BlockSpec((2 * NB, H), lambda i: (0, 0)),
                  pl.BlockSpec((2 * NB, H), lambda i: (0, 0)),
                  pl.BlockSpec((H, H), lambda i: (0, 0)),
                  pl.BlockSpec((1, H), lambda i: (0, 0))],
        out_specs=pl.BlockSpec((NB, H), lambda i: (0, 0)),
        out_shape=jax.ShapeDtypeStruct((NB, H), jnp.float32),
    )(sums, cnts, w2, b2)


def kernel(x_operator, x_table, x_column, ei_oo, ei_to, ei_co, ei_tt, ei_cc,
           batch_operator, lin_operator_w, lin_operator_b, lin_table_w,
           lin_table_b, lin_column_w, lin_column_b,
           w_rel_oo, b_rel_oo, w_root_oo, w_rel_to, b_rel_to, w_root_to,
           w_rel_co, b_rel_co, w_root_co, w_rel_tt, b_rel_tt, w_root_tt,
           w_rel_cc, b_rel_cc, w_root_cc,
           lin_mem_w, lin_mem_b, lin_time_w, lin_time_b):
    e_oo_p = _pad_len(ei_oo.shape[1])
    e_to_p = _pad_len(ei_to.shape[1])
    e_co_p = _pad_len(ei_co.shape[1])
    e_tt_p = _pad_len(ei_tt.shape[1])
    e_cc_p = _pad_len(ei_cc.shape[1])
    src_oo, dst_oo = _pad_edges(ei_oo, e_oo_p)
    src_to, dst_to = _pad_edges(ei_to, e_to_p)
    src_co, dst_co = _pad_edges(ei_co, e_co_p)
    src_tt, dst_tt = _pad_edges(ei_tt, e_tt_p)
    src_cc, dst_cc = _pad_edges(ei_cc, e_cc_p)

    zeros_h = jnp.zeros((SLAB, H), jnp.float32)
    ones_h = jnp.ones((SLAB, H), jnp.float32)
    batch_pad = jnp.concatenate(
        [batch_operator, jnp.full((OP_PAD - N_OP,), NB, jnp.int32)])

    w_root_op = w_root_oo + w_root_to + w_root_co
    b_op = b_rel_oo + b_rel_to + b_rel_co

    t_pad, c_pad = 5120, 20480
    agg_op = _make_agg(OP_PAD, ((e_oo_p, N_OP), (e_to_p, N_T), (e_co_p, N_C)))
    agg_t = _make_agg(t_pad, ((e_tt_p, N_T),))
    agg_c = _make_agg(c_pad, ((e_cc_p, N_C),))
    zeros_16 = jnp.zeros((16, H), jnp.float32)
    pool = _make_pool()

    x_op = _mm_bias(x_operator, lin_operator_w, lin_operator_b, n_out=OP_PAD)
    x_t = _mm_bias(x_table, lin_table_w, lin_table_b)
    x_c = _mm_bias(x_column, lin_column_w, lin_column_b)

    for _ in range(NUM_LAYERS):
        poo = _mm_plain(x_op, w_rel_oo)
        pto = _mm_plain(x_t, w_rel_to)
        pco = _mm_plain(x_c, w_rel_co)
        ptt = _mm_plain(x_t, w_rel_tt)
        pcc = _mm_plain(x_c, w_rel_cc)
        a_op = agg_op(zeros_16, src_oo, dst_oo, poo,
                      src_to, dst_to, pto,
                      src_co, dst_co, pco)
        a_t = agg_t(zeros_16, src_tt, dst_tt, ptt)
        a_c = agg_c(zeros_16, src_cc, dst_cc, pcc)
        x_op = _relu_root(a_op, x_op, w_root_op, b_op)
        x_t = _relu_root(a_t, x_t, w_root_tt, b_rel_tt)
        x_c = _relu_root(a_c, x_c, w_root_cc, b_rel_cc)

    sums, cnts = pool(x_op, batch_pad, zeros_h, ones_h)

    w2 = jnp.zeros((H, H), jnp.float32)
    w2 = w2.at[:, 0:1].set(lin_mem_w).at[:, 1:2].set(lin_time_w)
    b2 = jnp.zeros((1, H), jnp.float32)
    b2 = b2.at[0, 0].set(lin_mem_b[0]).at[0, 1].set(lin_time_b[0])
    out = _final_heads(sums, cnts, w2, b2)
    return (out[:, 0], out[:, 1])


# speedup vs baseline: 2.4183x; 1.0661x over previous
import functools

import jax
import jax.numpy as jnp
from jax import lax
from jax.experimental import pallas as pl
from jax.experimental.pallas import tpu as pltpu
from jax.experimental.pallas import tpu_sc as plsc

N_OP, N_T, N_C = 50000, 5000, 20000
H = 128
NB = 2048
NUM_LAYERS = 2
LANES = 16
SLAB = 128
EDGE_ALIGN = 32 * SLAB


def _pad_len(e):
    return ((e + EDGE_ALIGN - 1) // EDGE_ALIGN) * EDGE_ALIGN


def _pad_edges(ei, e_pad):
    e = ei.shape[1]
    src = jnp.concatenate([ei[0], jnp.zeros((e_pad - e,), jnp.int32)])
    dst = jnp.concatenate([ei[1], jnp.full((e_pad - e,), -1, jnp.int32)])
    return jnp.stack([src, dst])


def _dst_blocks(out_pad):
    max_rows = 12544
    nblk = -(-out_pad // max_rows)
    n128 = out_pad // 128
    blocks = []
    base = 0
    for i in range(nblk):
        size = (n128 - base // 128) // (nblk - i) * 128
        blocks.append((base, size))
        base += size
    return tuple(blocks)


def _make_agg(out_pad, rel_shapes):
    blocks = _dst_blocks(out_pad)
    zrows = ((max(b[1] for b in blocks) + 16 + 127) // 128) * 128
    zpt = zrows // 16
    trash = zrows - 8
    nrel = len(rel_shapes)
    chmax = max(e // 32 for e, _ in rel_shapes)
    mesh = plsc.VectorSubcoreMesh(core_axis_name="c", subcore_axis_name="s")

    out_t = jax.ShapeDtypeStruct((2 * out_pad, H), jnp.float32)
    scratch = [
        pltpu.VMEM((2, SLAB), jnp.int32),
        pltpu.VMEM((SLAB,), jnp.int32),
        pltpu.VMEM((SLAB, H), jnp.float32),
        pltpu.VMEM((64, H), jnp.float32),
        pltpu.VMEM_SHARED((zrows, H), jnp.float32),
        pltpu.SemaphoreType.DMA,
    ]

    @functools.partial(pl.kernel, out_type=out_t, scratch_types=scratch,
                       mesh=mesh)
    def agg(*refs):
        zeros_hbm = refs[0]
        rel_refs = refs[1:1 + 2 * nrel]
        o_hbm = refs[1 + 2 * nrel]
        (ed_st, dst_adj, rows, zbuf, acc, sem) = refs[2 + 2 * nrel:]
        c = lax.axis_index("c")
        s = lax.axis_index("s")
        trash_t = trash - 8 * s
        pltpu.sync_copy(zeros_hbm, zbuf)

        def run_rel(ed_hbm, p_hbm, chunk, base, bsize):
            nslab = chunk // SLAB
            base_e = (c * 16 + s) * chunk

            def body(j, carry, ed_hbm=ed_hbm,
                     p_hbm=p_hbm, chunk=chunk, base=base, bsize=bsize):
                off = base_e + j * SLAB
                pltpu.sync_copy(ed_hbm.at[:, pl.ds(off, SLAB)], ed_st)
                for k in range(SLAB // LANES):
                    d = ed_st[1, pl.ds(k * LANES, LANES)]
                    loc = d - base
                    ok = (loc >= 0) & (loc < bsize)
                    dst_adj[pl.ds(k * LANES, LANES)] = jnp.where(ok, loc,
                                                                 trash_t)
                pltpu.async_copy(p_hbm.at[ed_st.at[0]], rows, sem).wait()
                pltpu.sync_copy(rows, acc.at[dst_adj], add=True)
                return carry

            lax.fori_loop(0, nslab, body, None)

        for base, bsize in blocks:
            z0 = s * zpt
            nfull, zrem = zpt // 64, zpt % 64
            for q in range(nfull):
                pltpu.sync_copy(zbuf, acc.at[pl.ds(z0 + q * 64, 64)])
            if zrem:
                pltpu.sync_copy(zbuf.at[pl.ds(0, zrem)],
                                acc.at[pl.ds(z0 + nfull * 64, zrem)])
            plsc.subcore_barrier()
            for r, (e_pad, _n_src) in enumerate(rel_shapes):
                run_rel(rel_refs[2 * r + 0], rel_refs[2 * r + 1],
                        e_pad // 32, base, bsize)
            plsc.subcore_barrier()
            pti = bsize // 16
            obase = c * out_pad + base
            pltpu.sync_copy(acc.at[pl.ds(s * pti, pti)],
                            o_hbm.at[pl.ds(obase + s * pti, pti)])
            plsc.subcore_barrier()

    return agg


OP_PAD = 50176
_POOL_PT = OP_PAD // 32
_POOL_TAIL = _POOL_PT - (_POOL_PT // SLAB) * SLAB
_PACC = 2176


def _make_pool():
    mesh = plsc.VectorSubcoreMesh(core_axis_name="c", subcore_axis_name="s")
    out_t = [jax.ShapeDtypeStruct((2 * NB, H), jnp.float32),
             jax.ShapeDtypeStruct((2 * NB, H), jnp.float32)]
    zpt = _PACC // 16
    scratch = [
        pltpu.VMEM((SLAB,), jnp.int32),
        pltpu.VMEM((_POOL_TAIL,), jnp.int32),
        pltpu.VMEM((SLAB, H), jnp.float32),
        pltpu.VMEM((SLAB, H), jnp.float32),
        pltpu.VMEM((SLAB, H), jnp.float32),
        pltpu.VMEM_SHARED((_PACC, H), jnp.float32),
        pltpu.VMEM_SHARED((_PACC, H), jnp.float32),
        pltpu.SemaphoreType.DMA,
    ]

    @functools.partial(pl.kernel, out_type=out_t, scratch_types=scratch,
                       mesh=mesh)
    def pool(x_hbm, b_hbm, zeros_hbm, ones_hbm,
             sums, cnts,
             idx_v, idxt_v, rows, zbuf_h, ones_v,
             acc_s, acc_c, sem):
        c = lax.axis_index("c")
        s = lax.axis_index("s")
        pltpu.sync_copy(zeros_hbm, zbuf_h)
        pltpu.sync_copy(ones_hbm, ones_v)
        z0 = s * zpt
        for acc in (acc_s, acc_c):
            pltpu.sync_copy(zbuf_h, acc.at[pl.ds(z0, SLAB)])
            pltpu.sync_copy(zbuf_h.at[pl.ds(0, zpt - SLAB)],
                            acc.at[pl.ds(z0 + SLAB, zpt - SLAB)])
        plsc.subcore_barrier()
        tbase = (c * 16 + s) * _POOL_PT

        def body(j, carry):
            off = tbase + j * SLAB
            pltpu.sync_copy(b_hbm.at[pl.ds(off, SLAB)], idx_v)
            pltpu.sync_copy(x_hbm.at[pl.ds(off, SLAB)], rows)
            pltpu.sync_copy(rows, acc_s.at[idx_v], add=True)
            pltpu.sync_copy(ones_v, acc_c.at[idx_v], add=True)
            return carry

        lax.fori_loop(0, _POOL_PT // SLAB, body, None)
        toff = tbase + (_POOL_PT // SLAB) * SLAB
        pltpu.sync_copy(b_hbm.at[pl.ds(toff, _POOL_TAIL)], idxt_v)
        pltpu.sync_copy(x_hbm.at[pl.ds(toff, _POOL_TAIL)],
                        rows.at[pl.ds(0, _POOL_TAIL)])
        pltpu.sync_copy(rows.at[pl.ds(0, _POOL_TAIL)],
                        acc_s.at[idxt_v], add=True)
        pltpu.sync_copy(ones_v.at[pl.ds(0, _POOL_TAIL)],
                        acc_c.at[idxt_v], add=True)
        plsc.subcore_barrier()
        obase = c * NB + s * (NB // 16)
        pltpu.sync_copy(acc_s.at[pl.ds(s * (NB // 16), NB // 16)],
                        sums.at[pl.ds(obase, NB // 16)])
        pltpu.sync_copy(acc_c.at[pl.ds(s * (NB // 16), NB // 16)],
                        cnts.at[pl.ds(obase, NB // 16)])

    return pool


_PREC = lax.Precision.HIGHEST


def _mm_bias(x, w, b, n_out=None, bm=512):
    n, f = x.shape
    n_out = n if n_out is None else n_out

    def kfn(x_ref, w_ref, b_ref, o_ref):
        o_ref[...] = jnp.dot(x_ref[...], w_ref[...],
                             preferred_element_type=jnp.float32,
                             precision=_PREC) + b_ref[...]

    return pl.pallas_call(
        kfn,
        grid=(pl.cdiv(n_out, bm),),
        in_specs=[pl.BlockSpec((bm, f), lambda i: (i, 0)),
                  pl.BlockSpec((f, H), lambda i: (0, 0)),
                  pl.BlockSpec((1, H), lambda i: (0, 0))],
        out_specs=pl.BlockSpec((bm, H), lambda i: (i, 0)),
        out_shape=jax.ShapeDtypeStruct((n_out, H), jnp.float32),
    )(x, w, b.reshape(1, H))


def _mm_plain(x, w, bm=512):
    n, f = x.shape

    def kfn(x_ref, w_ref, o_ref):
        o_ref[...] = jnp.dot(x_ref[...], w_ref[...],
                             preferred_element_type=jnp.float32,
                             precision=_PREC)

    return pl.pallas_call(
        kfn,
        grid=(pl.cdiv(n, bm),),
        in_specs=[pl.BlockSpec((bm, f), lambda i: (i, 0)),
                  pl.BlockSpec((f, H), lambda i: (0, 0))],
        out_specs=pl.BlockSpec((bm, H), lambda i: (i, 0)),
        out_shape=jax.ShapeDtypeStruct((n, H), jnp.float32),
    )(x, w)


def _relu_root(a_all, x, w, b, n_out=None, bm=512):
    n = x.shape[0]
    n_out = n if n_out is None else n_out
    half = a_all.shape[0] // 2 // bm

    def kfn(a0_ref, a1_ref, x_ref, w_ref, b_ref, o_ref):
        o_ref[...] = jnp.maximum(
            a0_ref[...] + a1_ref[...]
            + jnp.dot(x_ref[...], w_ref[...],
                      preferred_element_type=jnp.float32,
                      precision=_PREC) + b_ref[...], 0.0)

    return pl.pallas_call(
        kfn,
        grid=(pl.cdiv(n_out, bm),),
        in_specs=[pl.BlockSpec((bm, H), lambda i: (i, 0)),
                  pl.BlockSpec((bm, H), lambda i, half=half: (i + half, 0)),
                  pl.BlockSpec((bm, H), lambda i: (i, 0)),
                  pl.BlockSpec((H, H), lambda i: (0, 0)),
                  pl.BlockSpec((1, H), lambda i: (0, 0))],
        out_specs=pl.BlockSpec((bm, H), lambda i: (i, 0)),
        out_shape=jax.ShapeDtypeStruct((n_out, H), jnp.float32),
    )(a_all, a_all, x, w, b.reshape(1, H))


def _final_heads(sums, cnts, w2, b2):
    def kfn(s_ref, c_ref, w_ref, b_ref, o_ref):
        cnt = c_ref[:NB, 0:1] + c_ref[NB:, 0:1]
        emb = (s_ref[:NB, :] + s_ref[NB:, :]) / jnp.maximum(cnt, 1.0)
        o_ref[...] = jnp.dot(emb, w_ref[...],
                             preferred_element_type=jnp.float32,
                             precision=_PREC) + b_ref[...]

    return pl.pallas_call(
        kfn,
        grid=(1,),
        in_specs=[pl.BlockSpec((2 * NB, H), lambda i: (0, 0)),
                  pl.BlockSpec((2 * NB, H), lambda i: (0, 0)),
                  pl.BlockSpec((H, H), lambda i: (0, 0)),
                  pl.BlockSpec((1, H), lambda i: (0, 0))],
        out_specs=pl.BlockSpec((NB, H), lambda i: (0, 0)),
        out_shape=jax.ShapeDtypeStruct((NB, H), jnp.float32),
    )(sums, cnts, w2, b2)


def kernel(x_operator, x_table, x_column, ei_oo, ei_to, ei_co, ei_tt, ei_cc,
           batch_operator, lin_operator_w, lin_operator_b, lin_table_w,
           lin_table_b, lin_column_w, lin_column_b,
           w_rel_oo, b_rel_oo, w_root_oo, w_rel_to, b_rel_to, w_root_to,
           w_rel_co, b_rel_co, w_root_co, w_rel_tt, b_rel_tt, w_root_tt,
           w_rel_cc, b_rel_cc, w_root_cc,
           lin_mem_w, lin_mem_b, lin_time_w, lin_time_b):
    e_oo_p = _pad_len(ei_oo.shape[1])
    e_to_p = _pad_len(ei_to.shape[1])
    e_co_p = _pad_len(ei_co.shape[1])
    e_tt_p = _pad_len(ei_tt.shape[1])
    e_cc_p = _pad_len(ei_cc.shape[1])
    ed_oo = _pad_edges(ei_oo, e_oo_p)
    ed_to = _pad_edges(ei_to, e_to_p)
    ed_co = _pad_edges(ei_co, e_co_p)
    ed_tt = _pad_edges(ei_tt, e_tt_p)
    ed_cc = _pad_edges(ei_cc, e_cc_p)

    zeros_h = jnp.zeros((SLAB, H), jnp.float32)
    ones_h = jnp.ones((SLAB, H), jnp.float32)
    batch_pad = jnp.concatenate(
        [batch_operator, jnp.full((OP_PAD - N_OP,), NB, jnp.int32)])

    w_root_op = w_root_oo + w_root_to + w_root_co
    b_op = b_rel_oo + b_rel_to + b_rel_co

    t_pad, c_pad = 5120, 20480
    agg_op = _make_agg(OP_PAD, ((e_oo_p, N_OP), (e_to_p, N_T), (e_co_p, N_C)))
    agg_t = _make_agg(t_pad, ((e_tt_p, N_T),))
    agg_c = _make_agg(c_pad, ((e_cc_p, N_C),))
    zeros_64 = jnp.zeros((64, H), jnp.float32)
    pool = _make_pool()

    x_op = _mm_bias(x_operator, lin_operator_w, lin_operator_b, n_out=OP_PAD)
    x_t = _mm_bias(x_table, lin_table_w, lin_table_b)
    x_c = _mm_bias(x_column, lin_column_w, lin_column_b)

    for _ in range(NUM_LAYERS):
        poo = _mm_plain(x_op, w_rel_oo)
        pto = _mm_plain(x_t, w_rel_to)
        pco = _mm_plain(x_c, w_rel_co)
        ptt = _mm_plain(x_t, w_rel_tt)
        pcc = _mm_plain(x_c, w_rel_cc)
        a_op = agg_op(zeros_64, ed_oo, poo, ed_to, pto, ed_co, pco)
        a_t = agg_t(zeros_64, ed_tt, ptt)
        a_c = agg_c(zeros_64, ed_cc, pcc)
        x_op = _relu_root(a_op, x_op, w_root_op, b_op)
        x_t = _relu_root(a_t, x_t, w_root_tt, b_rel_tt)
        x_c = _relu_root(a_c, x_c, w_root_cc, b_rel_cc)

    sums, cnts = pool(x_op, batch_pad, zeros_h, ones_h)

    w2 = jnp.zeros((H, H), jnp.float32)
    w2 = w2.at[:, 0:1].set(lin_mem_w).at[:, 1:2].set(lin_time_w)
    b2 = jnp.zeros((1, H), jnp.float32)
    b2 = b2.at[0, 0].set(lin_mem_b[0]).at[0, 1].set(lin_time_b[0])
    out = _final_heads(sums, cnts, w2, b2)
    return (out[:, 0], out[:, 1])
